# Initial kernel scaffold; baseline (speedup 1.0000x reference)
import jax
import jax.numpy as jnp
from jax.experimental import pallas as pl


def kernel(x, edge_index, edge_weight, W, b):
    raise NotImplementedError("write your pallas kernel here")


# speedup vs baseline: 7.2262x; 7.2262x over previous
import functools

import jax
import jax.numpy as jnp
from jax import lax
from jax.experimental import pallas as pl
from jax.experimental.pallas import tpu as pltpu
from jax.experimental.pallas import tpu_sc as plsc

N = 10000
D = 128
NC = 2
NS = 16
L = 16
NW = NC * NS
CHUNK = 128


def _matmul_kernel(x_ref, w_ref, b_ref, h_ref):
    h_ref[...] = (
        lax.dot_general(x_ref[...], w_ref[...], (((1,), (1,)), ((), ())),
                        preferred_element_type=jnp.float32)
        + b_ref[...]
    )


def _add_kernel(p_ref, o_ref):
    o_ref[...] = p_ref[0] + p_ref[1]


def _spmm_body(n_chunks, h_hbm, col_hbm, row_hbm, w_hbm, zeros_hbm, out_hbm,
               col_v, row_v, w_v, msgs, acc_smem, gsem):
    c = lax.axis_index("c")
    s = lax.axis_index("s")
    wid = c * NS + s

    pltpu.sync_copy(col_hbm.at[wid], col_v)
    pltpu.sync_copy(row_hbm.at[wid], row_v)
    pltpu.sync_copy(w_hbm.at[wid], w_v)

    n_row_chunks = N // 80
    pltpu.sync_copy(zeros_hbm, msgs)
    for i in range(-(-n_row_chunks // NS)):
        k = s + NS * i

        @pl.when(k < n_row_chunks)
        def _():
            pltpu.sync_copy(msgs.at[pl.ds(0, 80)],
                            acc_smem.at[pl.ds(k * 80, 80)])
    plsc.subcore_barrier()

    def chunk_body(j, _):
        pltpu.async_copy(h_hbm.at[col_v.at[j]], msgs, gsem).wait()
        for t in range(CHUNK // L):
            wrow = w_v[j, pl.ds(t * L, L)]
            for u in range(L):
                b = t * L + u
                wsplat = wrow[jnp.full((L,), u, jnp.int32)]
                for d in range(D // L):
                    sl = pl.ds(d * L, L)
                    msgs[b, sl] = msgs[b, sl] * wsplat
        pltpu.sync_copy(msgs, acc_smem.at[row_v.at[j]], add=True)
        return 0

    lax.fori_loop(0, n_chunks, chunk_body, 0)
    plsc.subcore_barrier()

    for i in range(-(-n_row_chunks // NS)):
        k = s + NS * i

        @pl.when(k < n_row_chunks)
        def _():
            pltpu.sync_copy(acc_smem.at[pl.ds(k * 80, 80)], msgs.at[pl.ds(0, 80)])
            pltpu.sync_copy(msgs.at[pl.ds(0, 80)],
                            out_hbm.at[c].at[pl.ds(k * 80, 80)])


def _spmm(h, col, row, w, n_chunks):
    mesh = plsc.VectorSubcoreMesh(
        core_axis_name="c", subcore_axis_name="s", num_cores=NC, num_subcores=NS)
    zeros = jnp.zeros((128, D), jnp.float32)
    kern = pl.kernel(
        functools.partial(_spmm_body, n_chunks),
        out_type=jax.ShapeDtypeStruct((NC, N, D), jnp.float32),
        mesh=mesh,
        scratch_types=[
            pltpu.VMEM((n_chunks, CHUNK), jnp.int32),
            pltpu.VMEM((n_chunks, CHUNK), jnp.int32),
            pltpu.VMEM((n_chunks, CHUNK), jnp.float32),
            pltpu.VMEM((CHUNK, D), jnp.float32),
            pltpu.VMEM_SHARED((N, D), jnp.float32),
            pltpu.SemaphoreType.DMA,
        ],
    )
    return kern(h, col, row, w, zeros)


def kernel(x, edge_index, edge_weight, W, b):
    E = edge_index.shape[1]
    n_chunks = -(-E // (NW * CHUNK))
    e_pad = NW * n_chunks * CHUNK

    pad = e_pad - E
    pad_idx = (jnp.arange(pad, dtype=jnp.int32) * 7) % N
    col = jnp.concatenate([edge_index[1], pad_idx]).reshape(NW, n_chunks, CHUNK)
    row = jnp.concatenate([edge_index[0], pad_idx]).reshape(NW, n_chunks, CHUNK)
    w = jnp.concatenate([edge_weight, jnp.zeros((pad,), jnp.float32)])
    w = w.reshape(NW, n_chunks, CHUNK)

    n_rows = x.shape[0]
    blk = 1000
    h = pl.pallas_call(
        _matmul_kernel,
        grid=(n_rows // blk,),
        in_specs=[
            pl.BlockSpec((blk, D), lambda i: (i, 0)),
            pl.BlockSpec((D, D), lambda i: (0, 0)),
            pl.BlockSpec((1, D), lambda i: (0, 0)),
        ],
        out_specs=pl.BlockSpec((blk, D), lambda i: (i, 0)),
        out_shape=jax.ShapeDtypeStruct((n_rows, D), jnp.float32),
    )(x, W, b.reshape(1, D))

    partials = _spmm(h, col, row, w, n_chunks)

    out = pl.pallas_call(
        _add_kernel,
        grid=(n_rows // blk,),
        in_specs=[pl.BlockSpec((NC, blk, D), lambda i: (0, i, 0))],
        out_specs=pl.BlockSpec((blk, D), lambda i: (i, 0)),
        out_shape=jax.ShapeDtypeStruct((n_rows, D), jnp.float32),
    )(partials)
    return out


# speedup vs baseline: 8.2195x; 1.1375x over previous
import functools

import jax
import jax.numpy as jnp
from jax import lax
from jax.experimental import pallas as pl
from jax.experimental.pallas import tpu as pltpu
from jax.experimental.pallas import tpu_sc as plsc

N = 10000
D = 128
NC = 2
NS = 16
L = 16
DH = D // NC
CHUNK = 128


def _matmul_kernel(x_ref, w_ref, b_ref, h_ref):
    h_ref[0] = (
        lax.dot_general(x_ref[...], w_ref[0], (((1,), (1,)), ((), ())),
                        preferred_element_type=jnp.float32)
        + b_ref[0]
    )


def _concat_kernel(p_ref, o_ref):
    o_ref[...] = jnp.concatenate([p_ref[0], p_ref[1]], axis=-1)


def _scale_rows(buf, w_v, j):
    for t in range(CHUNK // L):
        wrow = w_v[j, pl.ds(t * L, L)]
        for u in range(L):
            b = t * L + u
            wsplat = wrow[jnp.full((L,), u, jnp.int32)]
            for d in range(DH // L):
                sl = pl.ds(d * L, L)
                buf[b, sl] = buf[b, sl] * wsplat


def _spmm_body(n_chunks, h_hbm, col_hbm, row_hbm, w_hbm, zeros_hbm, out_hbm,
               col_v, row_v, w_v, buf0, buf1, buf2,
               acc_smem, gsem0, gsem1, gsem2, ssem0, ssem1, ssem2):
    c = lax.axis_index("c")
    s = lax.axis_index("s")
    bufs = (buf0, buf1, buf2)
    gsems = (gsem0, gsem1, gsem2)
    ssems = (ssem0, ssem1, ssem2)

    pltpu.sync_copy(col_hbm.at[s], col_v)
    pltpu.sync_copy(row_hbm.at[s], row_v)
    pltpu.sync_copy(w_hbm.at[s], w_v)

    n_row_chunks = N // 80
    pltpu.sync_copy(zeros_hbm, buf0)
    for i in range(-(-n_row_chunks // NS)):
        k = s + NS * i

        @pl.when(k < n_row_chunks)
        def _():
            pltpu.sync_copy(buf0.at[pl.ds(0, 80)],
                            acc_smem.at[pl.ds(k * 80, 80)])
    plsc.subcore_barrier()

    hc = h_hbm.at[c]
    pltpu.async_copy(hc.at[col_v.at[0]], buf0, gsem0)
    pltpu.async_copy(hc.at[col_v.at[1]], buf1, gsem1)

    def step(j, buf, prev_buf, gsem, prev_gsem, ssem, prev_ssem):
        pltpu.make_async_copy(hc.at[col_v.at[j]], buf, gsem).wait()
        _scale_rows(buf, w_v, j)
        pltpu.async_copy(buf, acc_smem.at[row_v.at[j]], ssem, add=True)

        @pl.when(j >= 1)
        def _():
            pltpu.make_async_copy(prev_buf, acc_smem.at[row_v.at[j - 1]],
                                  prev_ssem).wait()

        @pl.when(j + 2 < n_chunks)
        def _():
            pltpu.async_copy(hc.at[col_v.at[j + 2]], prev_buf, prev_gsem)

    def group_body(g, _):
        j = 3 * g
        step(j, buf0, buf2, gsem0, gsem2, ssem0, ssem2)
        step(j + 1, buf1, buf0, gsem1, gsem0, ssem1, ssem0)
        step(j + 2, buf2, buf1, gsem2, gsem1, ssem2, ssem1)
        return 0

    lax.fori_loop(0, n_chunks // 3, group_body, 0)

    pltpu.make_async_copy(buf2, acc_smem.at[row_v.at[n_chunks - 1]],
                          ssem2).wait()
    plsc.subcore_barrier()

    for i in range(-(-n_row_chunks // NS)):
        k = s + NS * i

        @pl.when(k < n_row_chunks)
        def _():
            pltpu.sync_copy(acc_smem.at[pl.ds(k * 80, 80)], buf0.at[pl.ds(0, 80)])
            pltpu.sync_copy(buf0.at[pl.ds(0, 80)],
                            out_hbm.at[c].at[pl.ds(k * 80, 80)])


def _spmm(h_split, col, row, w, n_chunks):
    mesh = plsc.VectorSubcoreMesh(
        core_axis_name="c", subcore_axis_name="s", num_cores=NC, num_subcores=NS)
    zeros = jnp.zeros((CHUNK, DH), jnp.float32)
    kern = pl.kernel(
        functools.partial(_spmm_body, n_chunks),
        out_type=jax.ShapeDtypeStruct((NC, N, DH), jnp.float32),
        mesh=mesh,
        compiler_params=pltpu.CompilerParams(use_tc_tiling_on_sc=False),
        scratch_types=[
            pltpu.VMEM((n_chunks, CHUNK), jnp.int32),
            pltpu.VMEM((n_chunks, CHUNK), jnp.int32),
            pltpu.VMEM((n_chunks, CHUNK), jnp.float32),
            pltpu.VMEM((CHUNK, DH), jnp.float32),
            pltpu.VMEM((CHUNK, DH), jnp.float32),
            pltpu.VMEM((CHUNK, DH), jnp.float32),
            pltpu.VMEM_SHARED((N, DH), jnp.float32),
            pltpu.SemaphoreType.DMA,
            pltpu.SemaphoreType.DMA,
            pltpu.SemaphoreType.DMA,
            pltpu.SemaphoreType.DMA,
            pltpu.SemaphoreType.DMA,
            pltpu.SemaphoreType.DMA,
        ],
    )
    return kern(h_split, col, row, w, zeros)


def kernel(x, edge_index, edge_weight, W, b):
    E = edge_index.shape[1]
    n_chunks = -(-E // (NS * CHUNK))
    n_chunks += (-n_chunks) % 3
    e_pad = NS * n_chunks * CHUNK

    pad = e_pad - E
    pad_idx = (jnp.arange(pad, dtype=jnp.int32) * 7) % N
    col = jnp.concatenate([edge_index[1], pad_idx]).reshape(NS, n_chunks, CHUNK)
    row = jnp.concatenate([edge_index[0], pad_idx]).reshape(NS, n_chunks, CHUNK)
    w = jnp.concatenate([edge_weight, jnp.zeros((pad,), jnp.float32)])
    w = w.reshape(NS, n_chunks, CHUNK)

    n_rows = x.shape[0]
    blk = 1000
    h_split = pl.pallas_call(
        _matmul_kernel,
        grid=(NC, n_rows // blk),
        in_specs=[
            pl.BlockSpec((blk, D), lambda c, i: (i, 0)),
            pl.BlockSpec((1, DH, D), lambda c, i: (c, 0, 0)),
            pl.BlockSpec((1, 1, DH), lambda c, i: (c, 0, 0)),
        ],
        out_specs=pl.BlockSpec((1, blk, DH), lambda c, i: (c, i, 0)),
        out_shape=jax.ShapeDtypeStruct((NC, n_rows, DH), jnp.float32),
    )(x, W.reshape(NC, DH, D), b.reshape(NC, 1, DH))

    parts = _spmm(h_split, col, row, w, n_chunks)

    out = pl.pallas_call(
        _concat_kernel,
        grid=(n_rows // blk,),
        in_specs=[pl.BlockSpec((NC, blk, DH), lambda i: (0, i, 0))],
        out_specs=pl.BlockSpec((blk, D), lambda i: (i, 0)),
        out_shape=jax.ShapeDtypeStruct((n_rows, D), jnp.float32),
    )(parts)
    return out
